# baseline (speedup 1.0000x reference)
import jax
import jax.numpy as jnp
from jax import lax
from jax.experimental import pallas as pl
from jax.experimental.pallas import tpu as pltpu
from jax.experimental.pallas import tpu_sc as plsc

N = 10000
E = 320000
D = 128
NC2 = 2
NCORE = 2
NSUB = 16
NW = NCORE * NSUB
EPW = E // NW
CH = 128
NCHUNK = 80
HCH = NCHUNK // 2
EPT = NCHUNK * CH
HEPT = HCH * CH
EPAD = NW * EPT
NACC = 10112
RPT = NACC // NSUB
RCHUNKS = [(0, 88), (88, 88), (176, 88), (264, 88), (352, 88), (440, 88),
           (528, 88), (616, 16)]
RZ = 88

_f32 = jnp.float32
_i32 = jnp.int32


def _sc_mesh():
    return plsc.VectorSubcoreMesh(core_axis_name="c", subcore_axis_name="s",
                                  num_cores=NCORE, num_subcores=NSUB)


def _zero_acc(sid, gbuf, zc_hbm, acc_sh):
    zb = gbuf.at[pl.ds(0, RZ)]
    pltpu.sync_copy(zc_hbm, zb)
    for off, sz in RCHUNKS:
        pltpu.sync_copy(gbuf.at[pl.ds(0, sz)],
                        acc_sh.at[pl.ds(sid * RPT + off, sz)])


def _drain_acc(sid, cid, gbuf, acc_sh, out0, out1):
    for off, sz in RCHUNKS:
        o = sid * RPT + off
        pltpu.sync_copy(acc_sh.at[pl.ds(o, sz)], gbuf.at[pl.ds(0, sz)])

        @pl.when(cid == 0)
        def _():
            pltpu.sync_copy(gbuf.at[pl.ds(0, sz)], out0.at[pl.ds(o, sz)])

        @pl.when(cid == 1)
        def _():
            pltpu.sync_copy(gbuf.at[pl.ds(0, sz)], out1.at[pl.ds(o, sz)])


def _agg_body(h_hbm, src_hbm, dst_hbm, zc_hbm, out0, out1,
              src_v, dst_v, gbuf, agg_sh, gsem0, gsem1, ssem0, ssem1):
    cid = lax.axis_index("c")
    sid = lax.axis_index("s")
    wid = cid * NSUB + sid

    _zero_acc(sid, gbuf.at[0], zc_hbm, agg_sh)
    pltpu.sync_copy(dst_hbm.at[wid], dst_v)

    plsc.subcore_barrier()

    def buf(b):
        return gbuf.at[b]

    def gsem(b):
        return gsem0 if b == 0 else gsem1

    def ssem(b):
        return ssem0 if b == 0 else ssem1

    def start_gather(c, b):
        pltpu.async_copy(h_hbm.at[src_v.at[pl.ds(c * CH, CH)]], buf(b),
                         gsem(b))

    def wait_gather(b):
        pltpu.make_async_copy(h_hbm.at[src_v.at[pl.ds(0, CH)]], buf(b),
                              gsem(b)).wait()

    def start_scatter(j, b):
        pltpu.async_copy(buf(b), agg_sh.at[dst_v.at[j]], ssem(b), add=True)

    def wait_scatter(b):
        pltpu.make_async_copy(buf(b), agg_sh.at[dst_v.at[0]],
                              ssem(b)).wait()

    for hb in range(2):
        pltpu.sync_copy(src_hbm.at[pl.ds(wid * EPT + hb * HEPT, HEPT)],
                        src_v)
        base = hb * HCH
        start_gather(0, 0)
        wait_gather(0)
        start_scatter(base, 0)
        start_gather(1, 1)

        def body(jo, carry):
            for b in range(2):
                c = 1 + jo * 2 + b
                bb = 1 - b
                wait_gather(bb)
                start_scatter(base + c, bb)
                wait_scatter(1 - bb)
                start_gather(c + 1, 1 - bb)
            return carry

        lax.fori_loop(0, (HCH - 2) // 2, body, 0)
        wait_gather(1)
        start_scatter(base + HCH - 1, 1)
        wait_scatter(0)
        wait_scatter(1)

    plsc.subcore_barrier()
    _drain_acc(sid, cid, gbuf.at[0], agg_sh, out0, out1)


def _deg_body(dst_hbm, zc_hbm, oc_hbm, out0, out1, dst_v, gbuf, deg_sh, gsem):
    cid = lax.axis_index("c")
    sid = lax.axis_index("s")
    wid = cid * NSUB + sid

    _zero_acc(sid, gbuf, zc_hbm, deg_sh)
    pltpu.sync_copy(dst_hbm.at[wid], dst_v)
    pltpu.sync_copy(oc_hbm, gbuf)

    plsc.subcore_barrier()

    def chunk(j, carry):
        pltpu.async_copy(gbuf, deg_sh.at[dst_v.at[j]], gsem, add=True)
        return carry

    lax.fori_loop(0, NCHUNK, chunk, 0)

    def drain(j, carry):
        pltpu.make_async_copy(gbuf, deg_sh.at[dst_v.at[0]], gsem).wait()
        return carry

    lax.fori_loop(0, NCHUNK, drain, 0)

    plsc.subcore_barrier()
    _drain_acc(sid, cid, gbuf, deg_sh, out0, out1)


def _make_agg():
    return pl.kernel(
        _agg_body,
        out_type=(jax.ShapeDtypeStruct((NACC, D), _f32),
                  jax.ShapeDtypeStruct((NACC, D), _f32)),
        mesh=_sc_mesh(),
        scratch_types=[
            pltpu.VMEM((HEPT,), _i32),
            pltpu.VMEM((NCHUNK, CH), _i32),
            pltpu.VMEM((2, CH, D), _f32),
            pltpu.VMEM_SHARED((NACC, D), _f32),
            pltpu.SemaphoreType.DMA,
            pltpu.SemaphoreType.DMA,
            pltpu.SemaphoreType.DMA,
            pltpu.SemaphoreType.DMA,
        ],
        name="sage_agg",
    )


def _make_deg():
    return pl.kernel(
        _deg_body,
        out_type=(jax.ShapeDtypeStruct((NACC, D), _f32),
                  jax.ShapeDtypeStruct((NACC, D), _f32)),
        mesh=_sc_mesh(),
        scratch_types=[
            pltpu.VMEM((NCHUNK, CH), _i32),
            pltpu.VMEM((CH, D), _f32),
            pltpu.VMEM_SHARED((NACC, D), _f32),
            pltpu.SemaphoreType.DMA,
        ],
        name="sage_deg",
    )


def _recip_body(d0_ref, d1_ref, r_ref):
    r_ref[...] = 1.0 / jnp.maximum(d0_ref[:N] + d1_ref[:N], 1.0)


def _node_apply(h, a0, a1, r, w1, w2, b, g, be):
    c = (a0[:N] + a1[:N]) * r
    z = (jnp.dot(h, w1, preferred_element_type=_f32)
         + jnp.dot(c, w2, preferred_element_type=_f32) + b)
    nrm = jnp.sqrt(jnp.sum(z * z, axis=1, keepdims=True))
    z = z / jnp.maximum(nrm, 1e-12)
    hh = jnp.maximum(z, 0.0)
    mean = jnp.mean(hh, axis=0, keepdims=True)
    ctr = hh - mean
    var = jnp.mean(ctr * ctr, axis=0, keepdims=True)
    return h + g * ctr * lax.rsqrt(var + 1e-5) + be


def _dense_body(h_ref, a0_ref, a1_ref, r_ref, w1_ref, w2_ref, b_ref, g_ref,
                be_ref, o_ref):
    o_ref[...] = _node_apply(h_ref[...], a0_ref[...], a1_ref[...], r_ref[...],
                             w1_ref[...], w2_ref[...], b_ref[...], g_ref[...],
                             be_ref[...])


def _dense_pred_body(h_ref, a0_ref, a1_ref, r_ref, w1_ref, w2_ref, b_ref,
                     g_ref, be_ref, wp_ref, bp_ref, o_ref, pq_ref):
    o = _node_apply(h_ref[...], a0_ref[...], a1_ref[...], r_ref[...],
                    w1_ref[...], w2_ref[...], b_ref[...], g_ref[...],
                    be_ref[...])
    o_ref[...] = o
    pq_ref[...] = jnp.dot(o, wp_ref[...], preferred_element_type=_f32) + bp_ref[...]


def _score_body(pq_hbm, src_hbm, dst_hbm, out_hbm, tab_v, src_v, dst_v, ob_v):
    cid = lax.axis_index("c")
    sid = lax.axis_index("s")
    wid = cid * NSUB + sid
    pltpu.sync_copy(pq_hbm, tab_v)
    pltpu.sync_copy(src_hbm.at[pl.ds(wid * EPW, EPW)], src_v)
    pltpu.sync_copy(dst_hbm.at[pl.ds(wid * EPW, EPW)], dst_v)

    def body(gi, carry):
        s4 = src_v[pl.ds(gi * 16, 16)] * 4
        t4 = dst_v[pl.ds(gi * 16, 16)] * 4
        p0 = plsc.load_gather(tab_v, [s4])
        p1 = plsc.load_gather(tab_v, [s4 + 1])
        q0 = plsc.load_gather(tab_v, [t4 + 2])
        q1 = plsc.load_gather(tab_v, [t4 + 3])
        ob_v[0, pl.ds(gi * 16, 16)] = p0 + q0
        ob_v[1, pl.ds(gi * 16, 16)] = p1 + q1
        return carry

    lax.fori_loop(0, EPW // 16, body, 0)
    pltpu.sync_copy(ob_v, out_hbm.at[wid])


def kernel(h, edge_index, W0, b0, gamma0, beta0, W1, b1, gamma1, beta1,
           W2, b2, gamma2, beta2, W3, b3, gamma3, beta3, W_pred, b_pred):
    src = edge_index[0]
    dst = edge_index[1]
    pad_iota = jnp.arange(EPAD - E, dtype=jnp.int32)
    srcF = jnp.concatenate([src, pad_iota % N])
    dstT = jnp.concatenate([dst, N + pad_iota % (NACC - N)]).reshape(
        NW, NCHUNK, CH)
    zc = jnp.zeros((RZ, D), _f32)
    oc = jnp.ones((CH, D), _f32)

    agg_fn = _make_agg()

    d0, d1 = _make_deg()(dstT, zc, oc)
    r = pl.pallas_call(
        _recip_body,
        out_shape=jax.ShapeDtypeStruct((N, D), _f32),
    )(d0, d1)

    dense_fn = pl.pallas_call(
        _dense_body,
        out_shape=jax.ShapeDtypeStruct((N, D), _f32),
    )
    wp = jnp.concatenate([W_pred[:D], W_pred[D:]], axis=1)
    bp = jnp.concatenate([b_pred, jnp.zeros((2,), _f32)]).reshape(1, 4)
    dense_pred_fn = pl.pallas_call(
        _dense_pred_body,
        out_shape=(jax.ShapeDtypeStruct((N, D), _f32),
                   jax.ShapeDtypeStruct((N, 4), _f32)),
    )

    params = [(W0, b0, gamma0, beta0), (W1, b1, gamma1, beta1),
              (W2, b2, gamma2, beta2), (W3, b3, gamma3, beta3)]
    pq = None
    for l, (W, b, g, be) in enumerate(params):
        a0, a1 = agg_fn(h, srcF, dstT, zc)
        args = (h, a0, a1, r, W[:D], W[D:], b.reshape(1, D),
                g.reshape(1, D), be.reshape(1, D))
        if l < 3:
            h = dense_fn(*args)
        else:
            h, pq = dense_pred_fn(*(args + (wp, bp)))

    score_fn = pl.kernel(
        _score_body,
        out_type=jax.ShapeDtypeStruct((NW, 2, EPW), _f32),
        mesh=_sc_mesh(),
        scratch_types=[
            pltpu.VMEM((N * 4,), _f32),
            pltpu.VMEM((EPW,), _i32),
            pltpu.VMEM((EPW,), _i32),
            pltpu.VMEM((2, EPW), _f32),
        ],
        compiler_params=pltpu.CompilerParams(needs_layout_passes=False),
        name="edge_score",
    )
    sc3 = score_fn(pq.reshape(N * 4), src, dst)
    return sc3.transpose(0, 2, 1).reshape(E, 2)


# speedup vs baseline: 1.1330x; 1.1330x over previous
import jax
import jax.numpy as jnp
from jax import lax
from jax.experimental import pallas as pl
from jax.experimental.pallas import tpu as pltpu
from jax.experimental.pallas import tpu_sc as plsc

N = 10000
E = 320000
D = 128
NC2 = 2
NCORE = 2
NSUB = 16
NW = NCORE * NSUB
EPW = E // NW
CH = 128
NCHUNK = 80
HCH = NCHUNK // 2
EPT = NCHUNK * CH
HEPT = HCH * CH
EPAD = NW * EPT
NACC = 10112
RPT = NACC // NSUB
RCHUNKS = [(0, 88), (88, 88), (176, 88), (264, 88), (352, 88), (440, 88),
           (528, 88), (616, 16)]
RZ = 88

_f32 = jnp.float32
_i32 = jnp.int32


def _sc_mesh():
    return plsc.VectorSubcoreMesh(core_axis_name="c", subcore_axis_name="s",
                                  num_cores=NCORE, num_subcores=NSUB)


def _zero_acc(sid, gbuf, zc_hbm, acc_sh):
    zb = gbuf.at[pl.ds(0, RZ)]
    pltpu.sync_copy(zc_hbm, zb)
    for off, sz in RCHUNKS:
        pltpu.sync_copy(gbuf.at[pl.ds(0, sz)],
                        acc_sh.at[pl.ds(sid * RPT + off, sz)])


def _drain_acc(sid, cid, gbuf, acc_sh, out0, out1):
    for off, sz in RCHUNKS:
        o = sid * RPT + off
        pltpu.sync_copy(acc_sh.at[pl.ds(o, sz)], gbuf.at[pl.ds(0, sz)])

        @pl.when(cid == 0)
        def _():
            pltpu.sync_copy(gbuf.at[pl.ds(0, sz)], out0.at[pl.ds(o, sz)])

        @pl.when(cid == 1)
        def _():
            pltpu.sync_copy(gbuf.at[pl.ds(0, sz)], out1.at[pl.ds(o, sz)])


def _agg_body(h_hbm, src_hbm, dst_hbm, zc_hbm, out0, out1,
              src_v, dst_v, gbuf, agg_sh, gsem0, gsem1, ssem0, ssem1):
    cid = lax.axis_index("c")
    sid = lax.axis_index("s")
    wid = cid * NSUB + sid

    _zero_acc(sid, gbuf.at[0], zc_hbm, agg_sh)
    pltpu.sync_copy(dst_hbm.at[wid], dst_v)

    plsc.subcore_barrier()

    def buf(b):
        return gbuf.at[b]

    def gsem(b):
        return gsem0 if b == 0 else gsem1

    def ssem(b):
        return ssem0 if b == 0 else ssem1

    def start_gather(c, b):
        pltpu.async_copy(h_hbm.at[src_v.at[pl.ds(c * CH, CH)]], buf(b),
                         gsem(b))

    def wait_gather(b):
        pltpu.make_async_copy(h_hbm.at[src_v.at[pl.ds(0, CH)]], buf(b),
                              gsem(b)).wait()

    def start_scatter(j, b):
        pltpu.async_copy(buf(b), agg_sh.at[dst_v.at[j]], ssem(b), add=True)

    def wait_scatter(b):
        pltpu.make_async_copy(buf(b), agg_sh.at[dst_v.at[0]],
                              ssem(b)).wait()

    for hb in range(2):
        pltpu.sync_copy(src_hbm.at[pl.ds(wid * EPT + hb * HEPT, HEPT)],
                        src_v)
        base = hb * HCH
        start_gather(0, 0)
        wait_gather(0)
        start_scatter(base, 0)
        start_gather(1, 1)

        def body(jo, carry):
            for b in range(2):
                c = 1 + jo * 2 + b
                bb = 1 - b
                wait_scatter(1 - bb)
                start_gather(c + 1, 1 - bb)
                wait_gather(bb)
                start_scatter(base + c, bb)
            return carry

        lax.fori_loop(0, (HCH - 2) // 2, body, 0)
        wait_gather(1)
        start_scatter(base + HCH - 1, 1)
        wait_scatter(0)
        wait_scatter(1)

    plsc.subcore_barrier()
    _drain_acc(sid, cid, gbuf.at[0], agg_sh, out0, out1)


def _deg_body(dst_hbm, zc_hbm, oc_hbm, out0, out1, dst_v, gbuf, deg_sh, gsem):
    cid = lax.axis_index("c")
    sid = lax.axis_index("s")
    wid = cid * NSUB + sid

    _zero_acc(sid, gbuf, zc_hbm, deg_sh)
    pltpu.sync_copy(dst_hbm.at[wid], dst_v)
    pltpu.sync_copy(oc_hbm, gbuf)

    plsc.subcore_barrier()

    def chunk(j, carry):
        pltpu.async_copy(gbuf, deg_sh.at[dst_v.at[j]], gsem, add=True)
        return carry

    lax.fori_loop(0, NCHUNK, chunk, 0)

    def drain(j, carry):
        pltpu.make_async_copy(gbuf, deg_sh.at[dst_v.at[0]], gsem).wait()
        return carry

    lax.fori_loop(0, NCHUNK, drain, 0)

    plsc.subcore_barrier()
    _drain_acc(sid, cid, gbuf, deg_sh, out0, out1)


def _make_agg():
    return pl.kernel(
        _agg_body,
        out_type=(jax.ShapeDtypeStruct((NACC, D), _f32),
                  jax.ShapeDtypeStruct((NACC, D), _f32)),
        mesh=_sc_mesh(),
        scratch_types=[
            pltpu.VMEM((HEPT,), _i32),
            pltpu.VMEM((NCHUNK, CH), _i32),
            pltpu.VMEM((2, CH, D), _f32),
            pltpu.VMEM_SHARED((NACC, D), _f32),
            pltpu.SemaphoreType.DMA,
            pltpu.SemaphoreType.DMA,
            pltpu.SemaphoreType.DMA,
            pltpu.SemaphoreType.DMA,
        ],
        name="sage_agg",
    )


def _make_deg():
    return pl.kernel(
        _deg_body,
        out_type=(jax.ShapeDtypeStruct((NACC, D), _f32),
                  jax.ShapeDtypeStruct((NACC, D), _f32)),
        mesh=_sc_mesh(),
        scratch_types=[
            pltpu.VMEM((NCHUNK, CH), _i32),
            pltpu.VMEM((CH, D), _f32),
            pltpu.VMEM_SHARED((NACC, D), _f32),
            pltpu.SemaphoreType.DMA,
        ],
        name="sage_deg",
    )


def _recip_body(d0_ref, d1_ref, r_ref):
    r_ref[...] = 1.0 / jnp.maximum(d0_ref[:N] + d1_ref[:N], 1.0)


def _node_apply(h, a0, a1, r, w1, w2, b, g, be):
    c = (a0[:N] + a1[:N]) * r
    z = (jnp.dot(h, w1, preferred_element_type=_f32)
         + jnp.dot(c, w2, preferred_element_type=_f32) + b)
    nrm = jnp.sqrt(jnp.sum(z * z, axis=1, keepdims=True))
    z = z / jnp.maximum(nrm, 1e-12)
    hh = jnp.maximum(z, 0.0)
    mean = jnp.mean(hh, axis=0, keepdims=True)
    ctr = hh - mean
    var = jnp.mean(ctr * ctr, axis=0, keepdims=True)
    return h + g * ctr * lax.rsqrt(var + 1e-5) + be


def _dense_body(h_ref, a0_ref, a1_ref, r_ref, w1_ref, w2_ref, b_ref, g_ref,
                be_ref, o_ref):
    o_ref[...] = _node_apply(h_ref[...], a0_ref[...], a1_ref[...], r_ref[...],
                             w1_ref[...], w2_ref[...], b_ref[...], g_ref[...],
                             be_ref[...])


def _dense_pred_body(h_ref, a0_ref, a1_ref, r_ref, w1_ref, w2_ref, b_ref,
                     g_ref, be_ref, wp_ref, bp_ref, o_ref, pq_ref):
    o = _node_apply(h_ref[...], a0_ref[...], a1_ref[...], r_ref[...],
                    w1_ref[...], w2_ref[...], b_ref[...], g_ref[...],
                    be_ref[...])
    o_ref[...] = o
    pq_ref[...] = jnp.dot(o, wp_ref[...], preferred_element_type=_f32) + bp_ref[...]


def _score_body(pq_hbm, src_hbm, dst_hbm, out_hbm, tab_v, src_v, dst_v, ob_v):
    cid = lax.axis_index("c")
    sid = lax.axis_index("s")
    wid = cid * NSUB + sid
    pltpu.sync_copy(pq_hbm, tab_v)
    pltpu.sync_copy(src_hbm.at[pl.ds(wid * EPW, EPW)], src_v)
    pltpu.sync_copy(dst_hbm.at[pl.ds(wid * EPW, EPW)], dst_v)

    def body(gi, carry):
        s4 = src_v[pl.ds(gi * 16, 16)] * 4
        t4 = dst_v[pl.ds(gi * 16, 16)] * 4
        p0 = plsc.load_gather(tab_v, [s4])
        p1 = plsc.load_gather(tab_v, [s4 + 1])
        q0 = plsc.load_gather(tab_v, [t4 + 2])
        q1 = plsc.load_gather(tab_v, [t4 + 3])
        ob_v[0, pl.ds(gi * 16, 16)] = p0 + q0
        ob_v[1, pl.ds(gi * 16, 16)] = p1 + q1
        return carry

    lax.fori_loop(0, EPW // 16, body, 0)
    pltpu.sync_copy(ob_v, out_hbm.at[wid])


def kernel(h, edge_index, W0, b0, gamma0, beta0, W1, b1, gamma1, beta1,
           W2, b2, gamma2, beta2, W3, b3, gamma3, beta3, W_pred, b_pred):
    src = edge_index[0]
    dst = edge_index[1]
    pad_iota = jnp.arange(EPAD - E, dtype=jnp.int32)
    srcF = jnp.concatenate([src, pad_iota % N])
    dstT = jnp.concatenate([dst, N + pad_iota % (NACC - N)]).reshape(
        NW, NCHUNK, CH)
    zc = jnp.zeros((RZ, D), _f32)
    oc = jnp.ones((CH, D), _f32)

    agg_fn = _make_agg()

    d0, d1 = _make_deg()(dstT, zc, oc)
    r = pl.pallas_call(
        _recip_body,
        out_shape=jax.ShapeDtypeStruct((N, D), _f32),
    )(d0, d1)

    dense_fn = pl.pallas_call(
        _dense_body,
        out_shape=jax.ShapeDtypeStruct((N, D), _f32),
    )
    wp = jnp.concatenate([W_pred[:D], W_pred[D:]], axis=1)
    bp = jnp.concatenate([b_pred, jnp.zeros((2,), _f32)]).reshape(1, 4)
    dense_pred_fn = pl.pallas_call(
        _dense_pred_body,
        out_shape=(jax.ShapeDtypeStruct((N, D), _f32),
                   jax.ShapeDtypeStruct((N, 4), _f32)),
    )

    params = [(W0, b0, gamma0, beta0), (W1, b1, gamma1, beta1),
              (W2, b2, gamma2, beta2), (W3, b3, gamma3, beta3)]
    pq = None
    for l, (W, b, g, be) in enumerate(params):
        a0, a1 = agg_fn(h, srcF, dstT, zc)
        args = (h, a0, a1, r, W[:D], W[D:], b.reshape(1, D),
                g.reshape(1, D), be.reshape(1, D))
        if l < 3:
            h = dense_fn(*args)
        else:
            h, pq = dense_pred_fn(*(args + (wp, bp)))

    score_fn = pl.kernel(
        _score_body,
        out_type=jax.ShapeDtypeStruct((NW, 2, EPW), _f32),
        mesh=_sc_mesh(),
        scratch_types=[
            pltpu.VMEM((N * 4,), _f32),
            pltpu.VMEM((EPW,), _i32),
            pltpu.VMEM((EPW,), _i32),
            pltpu.VMEM((2, EPW), _f32),
        ],
        compiler_params=pltpu.CompilerParams(needs_layout_passes=False),
        name="edge_score",
    )
    sc3 = score_fn(pq.reshape(N * 4), src, dst)
    return sc3.transpose(0, 2, 1).reshape(E, 2)


# speedup vs baseline: 1.1419x; 1.0078x over previous
import jax
import jax.numpy as jnp
from jax import lax
from jax.experimental import pallas as pl
from jax.experimental.pallas import tpu as pltpu
from jax.experimental.pallas import tpu_sc as plsc

N = 10000
E = 320000
D = 128
NC2 = 2
NCORE = 2
NSUB = 16
NW = NCORE * NSUB
EPW = E // NW
CH = 128
NCHUNK = 80
HCH = NCHUNK // 2
EPT = NCHUNK * CH
HEPT = HCH * CH
EPAD = NW * EPT
NACC = 10112
RPT = NACC // NSUB
RCHUNKS = [(0, 88), (88, 88), (176, 88), (264, 88), (352, 88), (440, 88),
           (528, 88), (616, 16)]
RZ = 88

_f32 = jnp.float32
_i32 = jnp.int32


def _sc_mesh():
    return plsc.VectorSubcoreMesh(core_axis_name="c", subcore_axis_name="s",
                                  num_cores=NCORE, num_subcores=NSUB)


def _zero_acc(sid, gbuf, zc_hbm, acc_sh):
    zb = gbuf.at[pl.ds(0, RZ)]
    pltpu.sync_copy(zc_hbm, zb)
    for off, sz in RCHUNKS:
        pltpu.sync_copy(gbuf.at[pl.ds(0, sz)],
                        acc_sh.at[pl.ds(sid * RPT + off, sz)])


def _drain_acc(sid, cid, gbuf, acc_sh, out0, out1):
    for off, sz in RCHUNKS:
        o = sid * RPT + off
        pltpu.sync_copy(acc_sh.at[pl.ds(o, sz)], gbuf.at[pl.ds(0, sz)])

        @pl.when(cid == 0)
        def _():
            pltpu.sync_copy(gbuf.at[pl.ds(0, sz)], out0.at[pl.ds(o, sz)])

        @pl.when(cid == 1)
        def _():
            pltpu.sync_copy(gbuf.at[pl.ds(0, sz)], out1.at[pl.ds(o, sz)])


def _agg_body(h_hbm, src_hbm, dst_hbm, zc_hbm, out0, out1,
              src_v, dst_v, gbuf, agg_sh, gsem0, gsem1, ssem0, ssem1):
    cid = lax.axis_index("c")
    sid = lax.axis_index("s")
    wid = cid * NSUB + sid

    _zero_acc(sid, gbuf.at[0], zc_hbm, agg_sh)
    pltpu.sync_copy(dst_hbm.at[wid], dst_v)

    plsc.subcore_barrier()

    def buf(b):
        return gbuf.at[b]

    def gsem(b):
        return gsem0 if b == 0 else gsem1

    def ssem(b):
        return ssem0 if b == 0 else ssem1

    def start_gather(c, b):
        pltpu.async_copy(h_hbm.at[src_v.at[pl.ds(c * CH, CH)]], buf(b),
                         gsem(b))

    def wait_gather(b):
        pltpu.make_async_copy(h_hbm.at[src_v.at[pl.ds(0, CH)]], buf(b),
                              gsem(b)).wait()

    def start_scatter(j, b):
        pltpu.async_copy(buf(b), agg_sh.at[dst_v.at[j]], ssem(b), add=True)

    def wait_scatter(b):
        pltpu.make_async_copy(buf(b), agg_sh.at[dst_v.at[0]],
                              ssem(b)).wait()

    for hb in range(2):
        pltpu.sync_copy(src_hbm.at[pl.ds(wid * EPT + hb * HEPT, HEPT)],
                        src_v)
        base = hb * HCH
        start_gather(0, 0)
        wait_gather(0)
        start_scatter(base, 0)
        start_gather(1, 1)

        def body(jo, carry):
            for b in range(2):
                c = 1 + jo * 2 + b
                bb = 1 - b
                wait_scatter(1 - bb)
                start_gather(c + 1, 1 - bb)
                wait_gather(bb)
                start_scatter(base + c, bb)
            return carry

        lax.fori_loop(0, (HCH - 2) // 2, body, 0)
        wait_gather(1)
        start_scatter(base + HCH - 1, 1)
        wait_scatter(0)
        wait_scatter(1)

    plsc.subcore_barrier()
    _drain_acc(sid, cid, gbuf.at[0], agg_sh, out0, out1)


def _deg_body(dst_hbm, zc_hbm, oc_hbm, out0, out1, dst_v, gbuf, deg_sh, gsem):
    cid = lax.axis_index("c")
    sid = lax.axis_index("s")
    wid = cid * NSUB + sid

    _zero_acc(sid, gbuf, zc_hbm, deg_sh)
    pltpu.sync_copy(dst_hbm.at[wid], dst_v)
    pltpu.sync_copy(oc_hbm, gbuf)

    plsc.subcore_barrier()

    def chunk(j, carry):
        pltpu.async_copy(gbuf, deg_sh.at[dst_v.at[j]], gsem, add=True)
        return carry

    lax.fori_loop(0, NCHUNK, chunk, 0)

    def drain(j, carry):
        pltpu.make_async_copy(gbuf, deg_sh.at[dst_v.at[0]], gsem).wait()
        return carry

    lax.fori_loop(0, NCHUNK, drain, 0)

    plsc.subcore_barrier()
    _drain_acc(sid, cid, gbuf, deg_sh, out0, out1)


def _make_agg():
    return pl.kernel(
        _agg_body,
        out_type=(jax.ShapeDtypeStruct((NACC, D), _f32),
                  jax.ShapeDtypeStruct((NACC, D), _f32)),
        mesh=_sc_mesh(),
        scratch_types=[
            pltpu.VMEM((HEPT,), _i32),
            pltpu.VMEM((NCHUNK, CH), _i32),
            pltpu.VMEM((2, CH, D), _f32),
            pltpu.VMEM_SHARED((NACC, D), _f32),
            pltpu.SemaphoreType.DMA,
            pltpu.SemaphoreType.DMA,
            pltpu.SemaphoreType.DMA,
            pltpu.SemaphoreType.DMA,
        ],
        name="sage_agg",
    )


def _make_deg():
    return pl.kernel(
        _deg_body,
        out_type=(jax.ShapeDtypeStruct((NACC, D), _f32),
                  jax.ShapeDtypeStruct((NACC, D), _f32)),
        mesh=_sc_mesh(),
        scratch_types=[
            pltpu.VMEM((NCHUNK, CH), _i32),
            pltpu.VMEM((CH, D), _f32),
            pltpu.VMEM_SHARED((NACC, D), _f32),
            pltpu.SemaphoreType.DMA,
        ],
        name="sage_deg",
    )


def _recip_body(d0_ref, d1_ref, r_ref):
    r_ref[...] = 1.0 / jnp.maximum(d0_ref[:N] + d1_ref[:N], 1.0)


def _node_apply(h, a0, a1, r, w1, w2, b, g, be):
    c = (a0[:N] + a1[:N]) * r
    z = (jnp.dot(h, w1, preferred_element_type=_f32)
         + jnp.dot(c, w2, preferred_element_type=_f32) + b)
    nrm = jnp.sqrt(jnp.sum(z * z, axis=1, keepdims=True))
    z = z / jnp.maximum(nrm, 1e-12)
    hh = jnp.maximum(z, 0.0)
    mean = jnp.mean(hh, axis=0, keepdims=True)
    ctr = hh - mean
    var = jnp.mean(ctr * ctr, axis=0, keepdims=True)
    return h + g * ctr * lax.rsqrt(var + 1e-5) + be


def _dense_body(h_ref, a0_ref, a1_ref, r_ref, w1_ref, w2_ref, b_ref, g_ref,
                be_ref, o_ref):
    o_ref[...] = _node_apply(h_ref[...], a0_ref[...], a1_ref[...], r_ref[...],
                             w1_ref[...], w2_ref[...], b_ref[...], g_ref[...],
                             be_ref[...])


def _dense0_body(h_ref, a0_ref, a1_ref, d0_ref, d1_ref, w1_ref, w2_ref,
                 b_ref, g_ref, be_ref, o_ref, r_ref):
    r = 1.0 / jnp.maximum(d0_ref[:N] + d1_ref[:N], 1.0)
    r_ref[...] = r
    o_ref[...] = _node_apply(h_ref[...], a0_ref[...], a1_ref[...], r,
                             w1_ref[...], w2_ref[...], b_ref[...], g_ref[...],
                             be_ref[...])


def _dense_pred_body(h_ref, a0_ref, a1_ref, r_ref, w1_ref, w2_ref, b_ref,
                     g_ref, be_ref, wp_ref, bp_ref, o_ref, pq_ref):
    o = _node_apply(h_ref[...], a0_ref[...], a1_ref[...], r_ref[...],
                    w1_ref[...], w2_ref[...], b_ref[...], g_ref[...],
                    be_ref[...])
    o_ref[...] = o
    pq_ref[...] = jnp.dot(o, wp_ref[...], preferred_element_type=_f32) + bp_ref[...]


def _score_body(pq_hbm, src_hbm, dst_hbm, out_hbm, tab_v, src_v, dst_v, ob_v,
                tab_sh):
    cid = lax.axis_index("c")
    sid = lax.axis_index("s")
    wid = cid * NSUB + sid
    @pl.when(sid == 0)
    def _():
        pltpu.sync_copy(pq_hbm, tab_sh)

    pltpu.sync_copy(src_hbm.at[pl.ds(wid * EPW, EPW)], src_v)
    pltpu.sync_copy(dst_hbm.at[pl.ds(wid * EPW, EPW)], dst_v)
    plsc.subcore_barrier()
    pltpu.sync_copy(tab_sh, tab_v)

    def body(gi, carry):
        s4 = src_v[pl.ds(gi * 16, 16)] * 4
        t4 = dst_v[pl.ds(gi * 16, 16)] * 4
        p0 = plsc.load_gather(tab_v, [s4])
        p1 = plsc.load_gather(tab_v, [s4 + 1])
        q0 = plsc.load_gather(tab_v, [t4 + 2])
        q1 = plsc.load_gather(tab_v, [t4 + 3])
        ob_v[0, pl.ds(gi * 16, 16)] = p0 + q0
        ob_v[1, pl.ds(gi * 16, 16)] = p1 + q1
        return carry

    lax.fori_loop(0, EPW // 16, body, 0)
    pltpu.sync_copy(ob_v, out_hbm.at[wid])


def kernel(h, edge_index, W0, b0, gamma0, beta0, W1, b1, gamma1, beta1,
           W2, b2, gamma2, beta2, W3, b3, gamma3, beta3, W_pred, b_pred):
    src = edge_index[0]
    dst = edge_index[1]
    pad_iota = jnp.arange(EPAD - E, dtype=jnp.int32)
    srcF = jnp.concatenate([src, pad_iota % N])
    dstT = jnp.concatenate([dst, N + pad_iota % (NACC - N)]).reshape(
        NW, NCHUNK, CH)
    zc = jnp.zeros((RZ, D), _f32)
    oc = jnp.ones((CH, D), _f32)

    agg_fn = _make_agg()

    d0, d1 = _make_deg()(dstT, zc, oc)

    dense_fn = pl.pallas_call(
        _dense_body,
        out_shape=jax.ShapeDtypeStruct((N, D), _f32),
    )
    dense0_fn = pl.pallas_call(
        _dense0_body,
        out_shape=(jax.ShapeDtypeStruct((N, D), _f32),
                   jax.ShapeDtypeStruct((N, D), _f32)),
    )
    wp = jnp.concatenate([W_pred[:D], W_pred[D:]], axis=1)
    bp = jnp.concatenate([b_pred, jnp.zeros((2,), _f32)]).reshape(1, 4)
    dense_pred_fn = pl.pallas_call(
        _dense_pred_body,
        out_shape=(jax.ShapeDtypeStruct((N, D), _f32),
                   jax.ShapeDtypeStruct((N, 4), _f32)),
    )

    params = [(W0, b0, gamma0, beta0), (W1, b1, gamma1, beta1),
              (W2, b2, gamma2, beta2), (W3, b3, gamma3, beta3)]
    pq = None
    r = None
    for l, (W, b, g, be) in enumerate(params):
        a0, a1 = agg_fn(h, srcF, dstT, zc)
        tail = (W[:D], W[D:], b.reshape(1, D), g.reshape(1, D),
                be.reshape(1, D))
        if l == 0:
            h, r = dense0_fn(h, a0, a1, d0, d1, *tail)
        elif l < 3:
            h = dense_fn(h, a0, a1, r, *tail)
        else:
            h, pq = dense_pred_fn(h, a0, a1, r, *(tail + (wp, bp)))

    score_fn = pl.kernel(
        _score_body,
        out_type=jax.ShapeDtypeStruct((NW, 2, EPW), _f32),
        mesh=_sc_mesh(),
        scratch_types=[
            pltpu.VMEM((N * 4,), _f32),
            pltpu.VMEM((EPW,), _i32),
            pltpu.VMEM((EPW,), _i32),
            pltpu.VMEM((2, EPW), _f32),
            pltpu.VMEM_SHARED((N * 4,), _f32),
        ],
        compiler_params=pltpu.CompilerParams(needs_layout_passes=False),
        name="edge_score",
    )
    sc3 = score_fn(pq.reshape(N * 4), src, dst)
    return sc3.transpose(0, 2, 1).reshape(E, 2)
